# baseline (speedup 1.0000x reference)
import functools

import jax
import jax.numpy as jnp
import numpy as np
from jax import lax
from jax.experimental import pallas as pl
from jax.experimental.pallas import tpu as pltpu
from jax.experimental.pallas import tpu_sc as plsc

_B, _T = 2, 30
_BT = _B * _T
_N, _E = 10000, 160000
_DI, _H, _TD, _NCLS = 8, 64, 128, 2

_NC, _NS, _L = 2, 16, 16
_FPC = _BT // _NC
_NP = 10240
_SLC = _NP // _NS
_EPT = _E // _NS
_EPTP = 10240
_CH = 64
_CSEG = _CH // _L
_NCH = _EPTP // _CH

_BNC = np.float32(1.0 / np.sqrt(1.0 + 1e-5))


def _mesh():
    return plsc.VectorSubcoreMesh(core_axis_name="c", subcore_axis_name="s")


def _sc_degree(dst, ew):
    _G = 10
    _NGRP = _NCH // _G
    scratch = [
        pltpu.VMEM((_EPTP,), jnp.int32),
        pltpu.VMEM((_EPTP,), jnp.float32),
    ]
    scratch += [pltpu.VMEM((_CH,), jnp.int32) for _ in range(_G)]
    scratch += [pltpu.VMEM((_CH, _L), jnp.float32) for _ in range(_G)]
    scratch += [
        pltpu.VMEM((_CH, _L), jnp.float32),
        pltpu.SemaphoreType.DMA,
        pltpu.SemaphoreType.DMA,
        pltpu.VMEM_SHARED((_NP, _L), jnp.float32),
    ]

    @functools.partial(
        pl.kernel,
        out_type=jax.ShapeDtypeStruct((_BT, _NP, _L), jnp.float32),
        mesh=_mesh(),
        scratch_types=scratch,
        compiler_params=pltpu.CompilerParams(use_tc_tiling_on_sc=False),
    )
    def kd(dst_hbm, ew_hbm, deg_hbm, *refs):
        dstT, ewT = refs[0], refs[1]
        dstbs = list(refs[2:2 + _G])
        rowbs = list(refs[2 + _G:2 + 2 * _G])
        zb, semg, sems, deg_sh = refs[2 + 2 * _G:]
        c = lax.axis_index("c")
        s = lax.axis_index("s")
        zrow = jnp.zeros((_L,), jnp.float32)
        zrowi = jnp.zeros((_L,), jnp.int32)

        def zloop(i, _):
            zb[i, :] = zrow
            return 0
        lax.fori_loop(0, _CH, zloop, 0)

        def frame_body(fi, _):
            f = c * _FPC + fi
            ebase = f * _E + s * _EPT
            cp1 = pltpu.async_copy(dst_hbm.at[pl.ds(ebase, _EPT)],
                                   dstT.at[pl.ds(0, _EPT)], semg)
            cp2 = pltpu.async_copy(ew_hbm.at[pl.ds(ebase, _EPT)],
                                   ewT.at[pl.ds(0, _EPT)], semg)
            for u in range(_SLC // _CH):
                pltpu.sync_copy(zb, deg_sh.at[pl.ds(s * _SLC + u * _CH, _CH)])
            cp1.wait()
            cp2.wait()
            for v in range((_EPTP - _EPT) // _L):
                dstT[pl.ds(_EPT + v * _L, _L)] = zrowi
                ewT[pl.ds(_EPT + v * _L, _L)] = zrow
            plsc.subcore_barrier()

            def group(m, _):
                for p in range(_G):
                    base = (m * _G + p) * _CH

                    def stg(g, _, _p=p, _base=base):
                        dstbs[_p][pl.ds(g * _L, _L)] = dstT[pl.ds(_base + g * _L, _L)]
                        return 0
                    lax.fori_loop(0, _CSEG, stg, 0)

                    def bld(g, _, _p=p, _base=base):
                        wseg = ewT[pl.ds(_base + g * _L, _L)]
                        for t in range(_L):
                            rowbs[_p][g * _L + t, :] = jnp.full((_L,), wseg[t],
                                                                jnp.float32)
                        return 0
                    lax.fori_loop(0, _CSEG, bld, 0)
                hs = [pltpu.async_copy(rowbs[p], deg_sh.at[dstbs[p]], sems,
                                       add=True) for p in range(_G)]
                for h in hs:
                    h.wait()
                return 0
            lax.fori_loop(0, _NGRP, group, 0)
            plsc.subcore_barrier()
            pltpu.sync_copy(deg_sh.at[pl.ds(s * _SLC, _SLC)],
                            deg_hbm.at[f, pl.ds(s * _SLC, _SLC)])
            plsc.subcore_barrier()
            return 0
        lax.fori_loop(0, _FPC, frame_body, 0)

    return kd(dst, ew)


def _sc_propagate(table2, src, dst, ew, width):
    _G2 = 4
    _NG = _NCH // _G2
    _KU = _NG // 4
    nseg = width // _L
    rbytes = _CH * width * 4

    scratch = [
        pltpu.VMEM((_EPTP,), jnp.int32),
        pltpu.VMEM((_EPTP,), jnp.int32),
        pltpu.VMEM((_EPTP,), jnp.float32),
    ]
    scratch += [pltpu.VMEM((_CH,), jnp.int32) for _ in range(2 * _G2)]
    scratch += [pltpu.VMEM((_CH,), jnp.int32) for _ in range(4 * _G2)]
    scratch += [pltpu.VMEM((_CH, width), jnp.float32) for _ in range(2 * _G2)]
    scratch += [
        pltpu.SemaphoreType.DMA,
        pltpu.SemaphoreType.DMA,
        pltpu.SemaphoreType.DMA,
        pltpu.SemaphoreType.DMA,
        pltpu.SemaphoreType.DMA,
        pltpu.VMEM_SHARED((_NP, width), jnp.float32),
    ]

    @functools.partial(
        pl.kernel,
        out_type=jax.ShapeDtypeStruct((_BT, _NP, width), jnp.float32),
        mesh=_mesh(),
        scratch_types=scratch,
        compiler_params=pltpu.CompilerParams(use_tc_tiling_on_sc=False),
    )
    def kp(tab_hbm, src_hbm, dst_hbm, ew_hbm, acc_hbm, *refs):
        srcT, dstT, ewT = refs[0], refs[1], refs[2]
        o = 3
        glbs = [list(refs[o + s * _G2:o + (s + 1) * _G2]) for s in range(2)]
        o += 2 * _G2
        dstbs = [list(refs[o + d * _G2:o + (d + 1) * _G2]) for d in range(4)]
        o += 4 * _G2
        rowbs = [list(refs[o + s * _G2:o + (s + 1) * _G2]) for s in range(2)]
        o += 2 * _G2
        semg = [refs[o], refs[o + 1]]
        sems = [refs[o + 2], refs[o + 3]]
        semp = refs[o + 4]
        acc_sh = refs[o + 5]
        c = lax.axis_index("c")
        s_id = lax.axis_index("s")
        zrow = jnp.zeros((_L,), jnp.float32)
        zrowi = jnp.zeros((_L,), jnp.int32)

        def drain(buf, sem):
            pltpu.make_async_copy(tab_hbm.at[pl.ds(0, _CH)], buf, sem).wait()

        def stage(mg, fbase):
            for p in range(_G2):
                base = (mg * _G2 + p) * _CH

                def stg(g, _, _p=p, _base=base):
                    sv = srcT[pl.ds(_base + g * _L, _L)]
                    glbs[mg % 2][_p][pl.ds(g * _L, _L)] = sv + fbase
                    dstbs[mg % 4][_p][pl.ds(g * _L, _L)] = dstT[pl.ds(_base + g * _L, _L)]
                    return 0
                lax.fori_loop(0, _CSEG, stg, 0)

        def gissue(sset):
            return [pltpu.async_copy(tab_hbm.at[glbs[sset][p]], rowbs[sset][p],
                                     semg[sset]) for p in range(_G2)]

        def frame_body(fi, _):
            f = c * _FPC + fi
            ebase = f * _E + s_id * _EPT
            cp1 = pltpu.async_copy(src_hbm.at[pl.ds(ebase, _EPT)],
                                   srcT.at[pl.ds(0, _EPT)], semp)
            cp2 = pltpu.async_copy(dst_hbm.at[pl.ds(ebase, _EPT)],
                                   dstT.at[pl.ds(0, _EPT)], semp)
            cp3 = pltpu.async_copy(ew_hbm.at[pl.ds(ebase, _EPT)],
                                   ewT.at[pl.ds(0, _EPT)], semp)
            def zrb(i, _):
                for u in range(nseg):
                    rowbs[0][0][i, pl.ds(u * _L, _L)] = zrow
                return 0
            lax.fori_loop(0, _CH, zrb, 0)
            for u in range(_SLC // _CH):
                pltpu.sync_copy(rowbs[0][0],
                                acc_sh.at[pl.ds(s_id * _SLC + u * _CH, _CH)])
            cp1.wait()
            cp2.wait()
            cp3.wait()
            for v in range((_EPTP - _EPT) // _L):
                srcT[pl.ds(_EPT + v * _L, _L)] = zrowi
                dstT[pl.ds(_EPT + v * _L, _L)] = zrowi
                ewT[pl.ds(_EPT + v * _L, _L)] = zrow
            plsc.subcore_barrier()
            fbase = f * _N

            stage(0, fbase)
            stage(1, fbase)
            gissue(0)

            def kbody(k, _):
                for j in range(4):
                    sset = j % 2
                    m4k = 4 * k + j
                    for p in range(_G2):
                        drain(rowbs[sset][p], semg[sset])
                    for p in range(_G2):
                        gbase = m4k * _G2 + p

                        def scl(g, _, _p=p, _gbase=gbase):
                            wseg = ewT[pl.ds(_gbase * _CH + g * _L, _L)]
                            for t in range(_L):
                                bc = jnp.full((_L,), wseg[t], jnp.float32)
                                for u in range(nseg):
                                    v = rowbs[sset][_p][g * _L + t, pl.ds(u * _L, _L)]
                                    rowbs[sset][_p][g * _L + t, pl.ds(u * _L, _L)] = v * bc
                            return 0
                        lax.fori_loop(0, _CSEG, scl, 0)
                    for p in range(_G2):
                        pltpu.async_copy(rowbs[sset][p],
                                         acc_sh.at[dstbs[j][p]], sems[sset],
                                         add=True)
                    if j < 2:
                        for p in range(_G2):
                            def stg2(g, _, _p=p, _j=j):
                                base = ((4 * k + _j + 2) * _G2 + _p) * _CH
                                sv = srcT[pl.ds(base + g * _L, _L)]
                                glbs[_j % 2][_p][pl.ds(g * _L, _L)] = sv + fbase
                                dstbs[(_j + 2) % 4][_p][pl.ds(g * _L, _L)] = \
                                    dstT[pl.ds(base + g * _L, _L)]
                                return 0
                            lax.fori_loop(0, _CSEG, stg2, 0)
                    else:
                        @pl.when(k < _KU - 1)
                        def _(_j=j):
                            for p in range(_G2):
                                def stg2(g, _, _p=p, __j=_j):
                                    base = ((4 * k + __j + 2) * _G2 + _p) * _CH
                                    sv = srcT[pl.ds(base + g * _L, _L)]
                                    glbs[__j % 2][_p][pl.ds(g * _L, _L)] = sv + fbase
                                    dstbs[(__j + 2) % 4][_p][pl.ds(g * _L, _L)] = \
                                        dstT[pl.ds(base + g * _L, _L)]
                                    return 0
                                lax.fori_loop(0, _CSEG, stg2, 0)
                    s1 = 1 - sset
                    if j == 0:
                        @pl.when(k > 0)
                        def _():
                            for p in range(_G2):
                                drain(rowbs[s1][p], sems[s1])
                        for p in range(_G2):
                            pltpu.async_copy(tab_hbm.at[glbs[s1][p]],
                                             rowbs[s1][p], semg[s1])
                    elif j < 3:
                        for p in range(_G2):
                            drain(rowbs[s1][p], sems[s1])
                        for p in range(_G2):
                            pltpu.async_copy(tab_hbm.at[glbs[s1][p]],
                                             rowbs[s1][p], semg[s1])
                    else:
                        @pl.when(k < _KU - 1)
                        def _():
                            for p in range(_G2):
                                drain(rowbs[s1][p], sems[s1])
                            for p in range(_G2):
                                pltpu.async_copy(tab_hbm.at[glbs[s1][p]],
                                                 rowbs[s1][p], semg[s1])
                return 0
            lax.fori_loop(0, _KU, kbody, 0)
            for p in range(_G2):
                drain(rowbs[0][p], sems[0])
            for p in range(_G2):
                drain(rowbs[1][p], sems[1])
            plsc.subcore_barrier()
            pltpu.sync_copy(acc_sh.at[pl.ds(s_id * _SLC, _SLC)],
                            acc_hbm.at[f, pl.ds(s_id * _SLC, _SLC)])
            plsc.subcore_barrier()
            return 0
        lax.fori_loop(0, _FPC, frame_body, 0)

    return kp(table2, src, dst, ew)


_NBLK = 2000
_NNB = _N // _NBLK


def _tc_prescale(x, deg):
    def body(x_ref, deg_ref, xs_ref, dinv_ref):
        d = deg_ref[0][:, :1] + 1.0
        dv = jnp.where(d > 0.0, lax.rsqrt(jnp.abs(d)), 0.0)
        xs = dv * x_ref[0]
        xs_ref[0] = jnp.concatenate(
            [xs, jnp.zeros((_NBLK, _L - _DI), jnp.float32)], axis=-1)
        dinv_ref[0] = dv

    return pl.pallas_call(
        body,
        grid=(_BT, _NNB),
        in_specs=[
            pl.BlockSpec((1, _NBLK, _DI), lambda f, nb: (f, nb, 0)),
            pl.BlockSpec((1, _NBLK, _L), lambda f, nb: (f, nb, 0)),
        ],
        out_specs=[
            pl.BlockSpec((1, _NBLK, _L), lambda f, nb: (f, nb, 0)),
            pl.BlockSpec((1, _NBLK, 1), lambda f, nb: (f, nb, 0)),
        ],
        out_shape=[
            jax.ShapeDtypeStruct((_BT, _N, _L), jnp.float32),
            jax.ShapeDtypeStruct((_BT, _N, 1), jnp.float32),
        ],
    )(x, deg)


def _tc_layer1(x, s1, dinv3, w1t, b1, g1, be1, w2t):
    def body(x_ref, s1_ref, dv_ref, w1t_ref, b1_ref, g1_ref, be1_ref, w2t_ref,
             h2s_ref):
        dv = dv_ref[0]
        xv = x_ref[0]
        s1v = s1_ref[0][:, :_DI]
        out1 = dv * s1v + (dv * dv) * xv
        h1 = jnp.dot(out1, w1t_ref[...], preferred_element_type=jnp.float32)
        h1 = h1 + b1_ref[...]
        z1 = jax.nn.relu(h1 * (g1_ref[...] * _BNC) + be1_ref[...])
        h2 = jnp.dot(z1, w2t_ref[...], preferred_element_type=jnp.float32)
        h2s_ref[0] = dv * h2

    return pl.pallas_call(
        body,
        grid=(_BT, _NNB),
        in_specs=[
            pl.BlockSpec((1, _NBLK, _DI), lambda f, nb: (f, nb, 0)),
            pl.BlockSpec((1, _NBLK, _L), lambda f, nb: (f, nb, 0)),
            pl.BlockSpec((1, _NBLK, 1), lambda f, nb: (f, nb, 0)),
            pl.BlockSpec((_DI, _H), lambda f, nb: (0, 0)),
            pl.BlockSpec((1, _H), lambda f, nb: (0, 0)),
            pl.BlockSpec((1, _H), lambda f, nb: (0, 0)),
            pl.BlockSpec((1, _H), lambda f, nb: (0, 0)),
            pl.BlockSpec((_H, _H), lambda f, nb: (0, 0)),
        ],
        out_specs=pl.BlockSpec((1, _NBLK, _H), lambda f, nb: (f, nb, 0)),
        out_shape=jax.ShapeDtypeStruct((_BT, _N, _H), jnp.float32),
    )(x, s1, dinv3, w1t, b1, g1, be1, w2t)


def _tc_layer2(s2, h2s, dinv3, b2, g2, be2):
    def body(s2_ref, h2s_ref, dv_ref, b2_ref, g2_ref, be2_ref, emb_ref):
        nb = pl.program_id(1)

        @pl.when(nb == 0)
        def _():
            emb_ref[...] = jnp.zeros_like(emb_ref)

        dv = dv_ref[0]
        out2 = dv * (s2_ref[0] + h2s_ref[0]) + b2_ref[...]
        z2 = jax.nn.relu(out2 * (g2_ref[...] * _BNC) + be2_ref[...])
        emb_ref[...] += (jnp.sum(z2, axis=0) * np.float32(1.0 / _N))[None, None]

    return pl.pallas_call(
        body,
        grid=(_BT, _NNB),
        in_specs=[
            pl.BlockSpec((1, _NBLK, _H), lambda f, nb: (f, nb, 0)),
            pl.BlockSpec((1, _NBLK, _H), lambda f, nb: (f, nb, 0)),
            pl.BlockSpec((1, _NBLK, 1), lambda f, nb: (f, nb, 0)),
            pl.BlockSpec((1, _H), lambda f, nb: (0, 0)),
            pl.BlockSpec((1, _H), lambda f, nb: (0, 0)),
            pl.BlockSpec((1, _H), lambda f, nb: (0, 0)),
        ],
        out_specs=pl.BlockSpec((1, 1, _H), lambda f, nb: (f, 0, 0)),
        out_shape=jax.ShapeDtypeStruct((_BT, 1, _H), jnp.float32),
    )(s2, h2s, dinv3, b2, g2, be2)


def _tc_gru_head(hseq, wiht, whht, bih, bhh, wc1t, bc1, wc2t, bc2):
    def body(hs_ref, wih_ref, whh_ref, bih_ref, bhh_ref, wc1_ref, bc1_ref,
             wc2_ref, bc2_ref, out_ref):
        wih = wih_ref[...]
        whh = whh_ref[...]
        bih_v = bih_ref[...]
        bhh_v = bhh_ref[...]

        def cell(t, h):
            xt = hs_ref[t]
            gi = jnp.dot(xt, wih, preferred_element_type=jnp.float32) + bih_v
            gh = jnp.dot(h, whh, preferred_element_type=jnp.float32) + bhh_v
            r = jax.nn.sigmoid(gi[:, :_TD] + gh[:, :_TD])
            z = jax.nn.sigmoid(gi[:, _TD:2 * _TD] + gh[:, _TD:2 * _TD])
            n = jnp.tanh(gi[:, 2 * _TD:] + r * gh[:, 2 * _TD:])
            return (1.0 - z) * n + z * h

        h = lax.fori_loop(0, _T, cell, jnp.zeros((_B, _TD), jnp.float32))
        hc = jax.nn.relu(
            jnp.dot(h, wc1_ref[...], preferred_element_type=jnp.float32)
            + bc1_ref[...])
        out_ref[...] = (jnp.dot(hc, wc2_ref[...],
                                preferred_element_type=jnp.float32)
                        + bc2_ref[...])

    return pl.pallas_call(
        body,
        out_shape=jax.ShapeDtypeStruct((_B, _NCLS), jnp.float32),
    )(hseq, wiht, whht, bih, bhh, wc1t, bc1, wc2t, bc2)


def kernel(x, edge_index, edge_weight, W1, b1, W2, b2, g1, be1, g2, be2,
           Wih, Whh, bih, bhh, Wc1, bc1, Wc2, bc2):
    src = edge_index[:, 0, :].reshape(-1)
    dst = edge_index[:, 1, :].reshape(-1)
    ew1 = edge_weight.reshape(-1)

    deg = _sc_degree(dst, ew1)
    xs16, dinv3 = _tc_prescale(x, deg)
    s1 = _sc_propagate(xs16.reshape(_BT * _N, _L), src, dst, ew1, _L)
    h2s = _tc_layer1(x, s1, dinv3, W1.T, b1[None], g1[None], be1[None], W2.T)
    s2 = _sc_propagate(h2s.reshape(_BT * _N, _H), src, dst, ew1, _H)
    embs = _tc_layer2(s2, h2s, dinv3, b2[None], g2[None], be2[None])

    hseq = embs.reshape(_B, _T, _H).transpose(1, 0, 2)
    return _tc_gru_head(hseq, Wih.T, Whh.T, bih[None], bhh[None],
                        Wc1.T, bc1[None], Wc2.T, bc2[None])


# speedup vs baseline: 2.1446x; 2.1446x over previous
import functools

import jax
import jax.numpy as jnp
import numpy as np
from jax import lax
from jax.experimental import pallas as pl
from jax.experimental.pallas import tpu as pltpu
from jax.experimental.pallas import tpu_sc as plsc

_B, _T = 2, 30
_BT = _B * _T
_N, _E = 10000, 160000
_DI, _H, _TD, _NCLS = 8, 64, 128, 2

_NC, _NS, _L = 2, 16, 16
_FPC = _BT // _NC
_NP = 10240
_SLC = _NP // _NS
_EPT = _E // _NS
_EPTP = 10240
_CH = 128
_NCH = _EPTP // _CH

_BNC = np.float32(1.0 / np.sqrt(1.0 + 1e-5))


def _mesh():
    return plsc.VectorSubcoreMesh(core_axis_name="c", subcore_axis_name="s")


def _sc_degree(dst, ew):
    _G = 10
    _NGRP = _NCH // _G
    scratch = [
        pltpu.VMEM((_EPTP,), jnp.int32),
        pltpu.VMEM((_EPTP,), jnp.float32),
    ]
    scratch += [pltpu.VMEM((_CH,), jnp.int32) for _ in range(_G)]
    scratch += [pltpu.VMEM((_CH, _L), jnp.float32) for _ in range(_G)]
    scratch += [
        pltpu.VMEM((_CH, _L), jnp.float32),
        pltpu.SemaphoreType.DMA,
        pltpu.SemaphoreType.DMA,
        pltpu.VMEM_SHARED((_NP, _L), jnp.float32),
    ]

    @functools.partial(
        pl.kernel,
        out_type=jax.ShapeDtypeStruct((_BT, _NP, _L), jnp.float32),
        mesh=_mesh(),
        scratch_types=scratch,
        compiler_params=pltpu.CompilerParams(use_tc_tiling_on_sc=False),
    )
    def kd(dst_hbm, ew_hbm, deg_hbm, *refs):
        dstT, ewT = refs[0], refs[1]
        dstbs = list(refs[2:2 + _G])
        rowbs = list(refs[2 + _G:2 + 2 * _G])
        zb, semg, sems, deg_sh = refs[2 + 2 * _G:]
        c = lax.axis_index("c")
        s = lax.axis_index("s")
        zrow = jnp.zeros((_L,), jnp.float32)
        zrowi = jnp.zeros((_L,), jnp.int32)

        def zloop(i, _):
            zb[i, :] = zrow
            return 0
        lax.fori_loop(0, _CH, zloop, 0)

        def frame_body(fi, _):
            f = c * _FPC + fi
            ebase = f * _E + s * _EPT
            cp1 = pltpu.async_copy(dst_hbm.at[pl.ds(ebase, _EPT)],
                                   dstT.at[pl.ds(0, _EPT)], semg)
            cp2 = pltpu.async_copy(ew_hbm.at[pl.ds(ebase, _EPT)],
                                   ewT.at[pl.ds(0, _EPT)], semg)
            for u in range(_SLC // _CH):
                pltpu.sync_copy(zb, deg_sh.at[pl.ds(s * _SLC + u * _CH, _CH)])
            cp1.wait()
            cp2.wait()
            for v in range((_EPTP - _EPT) // _L):
                dstT[pl.ds(_EPT + v * _L, _L)] = zrowi
                ewT[pl.ds(_EPT + v * _L, _L)] = zrow
            plsc.subcore_barrier()

            def group(m, _):
                for p in range(_G):
                    base = (m * _G + p) * _CH

                    def stg(g, _, _p=p, _base=base):
                        dstbs[_p][pl.ds(g * _L, _L)] = dstT[pl.ds(_base + g * _L, _L)]
                        return 0
                    lax.fori_loop(0, 8, stg, 0)

                    def bld(g, _, _p=p, _base=base):
                        wseg = ewT[pl.ds(_base + g * _L, _L)]
                        for t in range(_L):
                            rowbs[_p][g * _L + t, :] = jnp.full((_L,), wseg[t],
                                                                jnp.float32)
                        return 0
                    lax.fori_loop(0, 8, bld, 0)
                hs = [pltpu.async_copy(rowbs[p], deg_sh.at[dstbs[p]], sems,
                                       add=True) for p in range(_G)]
                for h in hs:
                    h.wait()
                return 0
            lax.fori_loop(0, _NGRP, group, 0)
            plsc.subcore_barrier()
            pltpu.sync_copy(deg_sh.at[pl.ds(s * _SLC, _SLC)],
                            deg_hbm.at[f, pl.ds(s * _SLC, _SLC)])
            plsc.subcore_barrier()
            return 0
        lax.fori_loop(0, _FPC, frame_body, 0)

    return kd(dst, ew)


def _sc_propagate(table2, src, dst, ew, width):
    _G = 5 if width == _H else 10
    _NGRP = _NCH // _G
    scratch = [
        pltpu.VMEM((_EPTP,), jnp.int32),
        pltpu.VMEM((_EPTP,), jnp.int32),
        pltpu.VMEM((_EPTP,), jnp.float32),
    ]
    scratch += [pltpu.VMEM((_CH,), jnp.int32) for _ in range(_G)]
    scratch += [pltpu.VMEM((_CH,), jnp.int32) for _ in range(_G)]
    scratch += [pltpu.VMEM((_CH, width), jnp.float32) for _ in range(_G)]
    scratch += [
        pltpu.VMEM((_CH, width), jnp.float32),
        pltpu.SemaphoreType.DMA,
        pltpu.SemaphoreType.DMA,
        pltpu.VMEM_SHARED((_NP, width), jnp.float32),
        pltpu.VMEM_SHARED((_NP, width), jnp.float32),
    ]
    nseg = width // _L

    @functools.partial(
        pl.kernel,
        out_type=jax.ShapeDtypeStruct((_BT, _NP, width), jnp.float32),
        mesh=_mesh(),
        scratch_types=scratch,
        compiler_params=pltpu.CompilerParams(use_tc_tiling_on_sc=False),
    )
    def kp(tab_hbm, src_hbm, dst_hbm, ew_hbm, acc_hbm, *refs):
        srcT, dstT, ewT = refs[0], refs[1], refs[2]
        glbs = list(refs[3:3 + _G])
        dstbs = list(refs[3 + _G:3 + 2 * _G])
        rowbs = list(refs[3 + 2 * _G:3 + 3 * _G])
        zb, semg, sems, acc_sh, tab_sh = refs[3 + 3 * _G:]
        c = lax.axis_index("c")
        s = lax.axis_index("s")
        zrow = jnp.zeros((_L,), jnp.float32)
        zrowi = jnp.zeros((_L,), jnp.int32)

        def zloop(i, _):
            for u in range(nseg):
                zb[i, pl.ds(u * _L, _L)] = zrow
            return 0
        lax.fori_loop(0, _CH, zloop, 0)

        def frame_body(fi, _):
            f = c * _FPC + fi
            ebase = f * _E + s * _EPT
            cp1 = pltpu.async_copy(src_hbm.at[pl.ds(ebase, _EPT)],
                                   srcT.at[pl.ds(0, _EPT)], semg)
            cp2 = pltpu.async_copy(dst_hbm.at[pl.ds(ebase, _EPT)],
                                   dstT.at[pl.ds(0, _EPT)], semg)
            cp3 = pltpu.async_copy(ew_hbm.at[pl.ds(ebase, _EPT)],
                                   ewT.at[pl.ds(0, _EPT)], semg)
            for u in range(_SLC // _CH):
                pltpu.sync_copy(zb, acc_sh.at[pl.ds(s * _SLC + u * _CH, _CH)])
            pltpu.sync_copy(tab_hbm.at[f, pl.ds(s * _SLC, _SLC)],
                            tab_sh.at[pl.ds(s * _SLC, _SLC)])
            cp1.wait()
            cp2.wait()
            cp3.wait()
            for v in range((_EPTP - _EPT) // _L):
                srcT[pl.ds(_EPT + v * _L, _L)] = zrowi
                dstT[pl.ds(_EPT + v * _L, _L)] = zrowi
                ewT[pl.ds(_EPT + v * _L, _L)] = zrow
            plsc.subcore_barrier()

            def group(m, _):
                hg = []
                for p in range(_G):
                    base = (m * _G + p) * _CH

                    def stg(g, _, _p=p, _base=base):
                        glbs[_p][pl.ds(g * _L, _L)] = srcT[pl.ds(_base + g * _L, _L)]
                        dstbs[_p][pl.ds(g * _L, _L)] = dstT[pl.ds(_base + g * _L, _L)]
                        return 0
                    lax.fori_loop(0, 8, stg, 0)
                    hg.append(pltpu.async_copy(tab_sh.at[glbs[p]], rowbs[p], semg))
                hs = []
                for p in range(_G):
                    base = (m * _G + p) * _CH
                    hg[p].wait()

                    def scl(g, _, _p=p, _base=base):
                        wseg = ewT[pl.ds(_base + g * _L, _L)]
                        for t in range(_L):
                            bc = jnp.full((_L,), wseg[t], jnp.float32)
                            e = g * _L + t
                            for u in range(nseg):
                                v = rowbs[_p][e, pl.ds(u * _L, _L)]
                                rowbs[_p][e, pl.ds(u * _L, _L)] = v * bc
                        return 0
                    lax.fori_loop(0, 8, scl, 0)
                    hs.append(pltpu.async_copy(rowbs[p], acc_sh.at[dstbs[p]], sems,
                                               add=True))
                for h in hs:
                    h.wait()
                return 0
            lax.fori_loop(0, _NGRP, group, 0)
            plsc.subcore_barrier()
            pltpu.sync_copy(acc_sh.at[pl.ds(s * _SLC, _SLC)],
                            acc_hbm.at[f, pl.ds(s * _SLC, _SLC)])
            plsc.subcore_barrier()
            return 0
        lax.fori_loop(0, _FPC, frame_body, 0)

    return kp(table2, src, dst, ew)


_NBLK = 2000
_NNB = _N // _NBLK


def _tc_prescale(x, deg):
    def body(x_ref, deg_ref, xs_ref, dinv_ref):
        d = deg_ref[0][:, :1] + 1.0
        dv = jnp.where(d > 0.0, lax.rsqrt(jnp.abs(d)), 0.0)
        xs = dv * x_ref[0]
        xs_ref[0] = jnp.concatenate(
            [xs, jnp.zeros((_NBLK, _L - _DI), jnp.float32)], axis=-1)
        dinv_ref[0] = dv

    return pl.pallas_call(
        body,
        grid=(_BT, _NNB),
        in_specs=[
            pl.BlockSpec((1, _NBLK, _DI), lambda f, nb: (f, nb, 0)),
            pl.BlockSpec((1, _NBLK, _L), lambda f, nb: (f, nb, 0)),
        ],
        out_specs=[
            pl.BlockSpec((1, _NBLK, _L), lambda f, nb: (f, nb, 0)),
            pl.BlockSpec((1, _NBLK, 1), lambda f, nb: (f, nb, 0)),
        ],
        out_shape=[
            jax.ShapeDtypeStruct((_BT, _NP, _L), jnp.float32),
            jax.ShapeDtypeStruct((_BT, _N, 1), jnp.float32),
        ],
    )(x, deg)


def _tc_layer1(x, s1, dinv3, w1t, b1, g1, be1, w2t):
    def body(x_ref, s1_ref, dv_ref, w1t_ref, b1_ref, g1_ref, be1_ref, w2t_ref,
             ha_ref, hb_ref):
        dv = dv_ref[0]
        xv = x_ref[0]
        s1v = s1_ref[0][:, :_DI]
        out1 = dv * s1v + (dv * dv) * xv
        h1 = jnp.dot(out1, w1t_ref[...], preferred_element_type=jnp.float32)
        h1 = h1 + b1_ref[...]
        z1 = jax.nn.relu(h1 * (g1_ref[...] * _BNC) + be1_ref[...])
        h2 = jnp.dot(z1, w2t_ref[...], preferred_element_type=jnp.float32)
        h2s = dv * h2
        ha_ref[0] = h2s[:, :_H // 2]
        hb_ref[0] = h2s[:, _H // 2:]

    return pl.pallas_call(
        body,
        grid=(_BT, _NNB),
        in_specs=[
            pl.BlockSpec((1, _NBLK, _DI), lambda f, nb: (f, nb, 0)),
            pl.BlockSpec((1, _NBLK, _L), lambda f, nb: (f, nb, 0)),
            pl.BlockSpec((1, _NBLK, 1), lambda f, nb: (f, nb, 0)),
            pl.BlockSpec((_DI, _H), lambda f, nb: (0, 0)),
            pl.BlockSpec((1, _H), lambda f, nb: (0, 0)),
            pl.BlockSpec((1, _H), lambda f, nb: (0, 0)),
            pl.BlockSpec((1, _H), lambda f, nb: (0, 0)),
            pl.BlockSpec((_H, _H), lambda f, nb: (0, 0)),
        ],
        out_specs=[
            pl.BlockSpec((1, _NBLK, _H // 2), lambda f, nb: (f, nb, 0)),
            pl.BlockSpec((1, _NBLK, _H // 2), lambda f, nb: (f, nb, 0)),
        ],
        out_shape=[
            jax.ShapeDtypeStruct((_BT, _NP, _H // 2), jnp.float32),
            jax.ShapeDtypeStruct((_BT, _NP, _H // 2), jnp.float32),
        ],
    )(x, s1, dinv3, w1t, b1, g1, be1, w2t)


def _tc_layer2(s2a, s2b, h2sa, h2sb, dinv3, b2, g2, be2):
    def body(sa_ref, sb_ref, ha_ref, hb_ref, dv_ref, b2_ref, g2_ref, be2_ref,
             emb_ref):
        nb = pl.program_id(1)

        @pl.when(nb == 0)
        def _():
            emb_ref[...] = jnp.zeros_like(emb_ref)

        dv = dv_ref[0]
        out2 = jnp.concatenate(
            [dv * (sa_ref[0] + ha_ref[0]), dv * (sb_ref[0] + hb_ref[0])],
            axis=-1) + b2_ref[...]
        z2 = jax.nn.relu(out2 * (g2_ref[...] * _BNC) + be2_ref[...])
        emb_ref[...] += (jnp.sum(z2, axis=0) * np.float32(1.0 / _N))[None, None]

    return pl.pallas_call(
        body,
        grid=(_BT, _NNB),
        in_specs=[
            pl.BlockSpec((1, _NBLK, _H // 2), lambda f, nb: (f, nb, 0)),
            pl.BlockSpec((1, _NBLK, _H // 2), lambda f, nb: (f, nb, 0)),
            pl.BlockSpec((1, _NBLK, _H // 2), lambda f, nb: (f, nb, 0)),
            pl.BlockSpec((1, _NBLK, _H // 2), lambda f, nb: (f, nb, 0)),
            pl.BlockSpec((1, _NBLK, 1), lambda f, nb: (f, nb, 0)),
            pl.BlockSpec((1, _H), lambda f, nb: (0, 0)),
            pl.BlockSpec((1, _H), lambda f, nb: (0, 0)),
            pl.BlockSpec((1, _H), lambda f, nb: (0, 0)),
        ],
        out_specs=pl.BlockSpec((1, 1, _H), lambda f, nb: (f, 0, 0)),
        out_shape=jax.ShapeDtypeStruct((_BT, 1, _H), jnp.float32),
    )(s2a, s2b, h2sa, h2sb, dinv3, b2, g2, be2)


def _tc_gru_head(hseq, wiht, whht, bih, bhh, wc1t, bc1, wc2t, bc2):
    def body(hs_ref, wih_ref, whh_ref, bih_ref, bhh_ref, wc1_ref, bc1_ref,
             wc2_ref, bc2_ref, out_ref):
        wih = wih_ref[...]
        whh = whh_ref[...]
        bih_v = bih_ref[...]
        bhh_v = bhh_ref[...]

        def cell(t, h):
            xt = hs_ref[t]
            gi = jnp.dot(xt, wih, preferred_element_type=jnp.float32) + bih_v
            gh = jnp.dot(h, whh, preferred_element_type=jnp.float32) + bhh_v
            r = jax.nn.sigmoid(gi[:, :_TD] + gh[:, :_TD])
            z = jax.nn.sigmoid(gi[:, _TD:2 * _TD] + gh[:, _TD:2 * _TD])
            n = jnp.tanh(gi[:, 2 * _TD:] + r * gh[:, 2 * _TD:])
            return (1.0 - z) * n + z * h

        h = lax.fori_loop(0, _T, cell, jnp.zeros((_B, _TD), jnp.float32))
        hc = jax.nn.relu(
            jnp.dot(h, wc1_ref[...], preferred_element_type=jnp.float32)
            + bc1_ref[...])
        out_ref[...] = (jnp.dot(hc, wc2_ref[...],
                                preferred_element_type=jnp.float32)
                        + bc2_ref[...])

    return pl.pallas_call(
        body,
        out_shape=jax.ShapeDtypeStruct((_B, _NCLS), jnp.float32),
    )(hseq, wiht, whht, bih, bhh, wc1t, bc1, wc2t, bc2)


def kernel(x, edge_index, edge_weight, W1, b1, W2, b2, g1, be1, g2, be2,
           Wih, Whh, bih, bhh, Wc1, bc1, Wc2, bc2):
    src = edge_index[:, 0, :].reshape(-1)
    dst = edge_index[:, 1, :].reshape(-1)
    ew1 = edge_weight.reshape(-1)

    deg = _sc_degree(dst, ew1)
    xs16, dinv3 = _tc_prescale(x, deg)
    s1 = _sc_propagate(xs16, src, dst, ew1, _L)
    h2sa, h2sb = _tc_layer1(x, s1, dinv3, W1.T, b1[None], g1[None],
                            be1[None], W2.T)
    s2a = _sc_propagate(h2sa, src, dst, ew1, _H // 2)
    s2b = _sc_propagate(h2sb, src, dst, ew1, _H // 2)
    embs = _tc_layer2(s2a, s2b, h2sa, h2sb, dinv3, b2[None], g2[None],
                      be2[None])

    hseq = embs.reshape(_B, _T, _H).transpose(1, 0, 2)
    return _tc_gru_head(hseq, Wih.T, Whh.T, bih[None], bhh[None],
                        Wc1.T, bc1[None], Wc2.T, bc2[None])
